# Initial kernel scaffold; baseline (speedup 1.0000x reference)
import jax
import jax.numpy as jnp
from jax.experimental import pallas as pl


def kernel(img_lr, h_weight, d_weight, b_weight):
    raise NotImplementedError("write your pallas kernel here")


# speedup vs baseline: 470.7892x; 470.7892x over previous
import functools

import jax
import jax.numpy as jnp
from jax import lax
from jax.experimental import pallas as pl
from jax.experimental.pallas import tpu as pltpu
from jax.experimental.pallas import tpu_sc as plsc

L = 17
H = 512
W = 512
NIMG = 12
NWORK = 32
RPW = H // NWORK
BUFR = RPW + 4
OFF = 8
RS = 528
NG = W // 16

PERMS = {
    0: (0, 1, 2, 3),
    1: (2, 0, 3, 1),
    2: (3, 2, 1, 0),
    3: (1, 3, 0, 2),
}
BRANCHES = (
    (0, (0, 1), (0, 2), 0),
    (0, (1, 0), (2, 0), 1),
    (0, (0, -1), (0, -2), 2),
    (0, (-1, 0), (-2, 0), 3),
    (1, (1, 1), (2, 2), 0),
    (1, (1, -1), (2, -2), 1),
    (1, (-1, -1), (-2, -2), 2),
    (1, (-1, 1), (-2, 2), 3),
    (2, (1, 2), (2, 1), 0),
    (2, (2, -1), (1, -2), 1),
    (2, (-1, -2), (-2, -1), 2),
    (2, (-2, 1), (-1, 2), 3),
)

_mesh = plsc.VectorSubcoreMesh(core_axis_name="c", subcore_axis_name="s")


@functools.partial(
    pl.kernel,
    out_type=jax.ShapeDtypeStruct((NIMG, 2 * H, 2 * W), jnp.float32),
    mesh=_mesh,
    scratch_types=[
        pltpu.VMEM((L**3 * 4,), jnp.float32),
        pltpu.VMEM((L**3 * 4,), jnp.float32),
        pltpu.VMEM((L**3 * 4,), jnp.float32),
        pltpu.VMEM((BUFR, RS), jnp.float32),
        pltpu.VMEM((RPW, 2 * W), jnp.float32),
        pltpu.SemaphoreType.DMA,
    ],
    compiler_params=pltpu.CompilerParams(
        use_tc_tiling_on_sc=False, needs_layout_passes=False),
)
def _hdblut_sc(img_hbm, hw_hbm, dw_hbm, bw_hbm, out_hbm,
               htab, dtab, btab, rows, outb, sem):
    wid = lax.axis_index("c") * 16 + lax.axis_index("s")
    s0 = wid * RPW

    pltpu.sync_copy(hw_hbm, htab)
    pltpu.sync_copy(dw_hbm, dtab)
    pltpu.sync_copy(bw_hbm, btab)

    iota = lax.iota(jnp.int32, 16)
    iota2 = iota * 2
    lpad_cols = jnp.maximum(iota - 2, 0) + OFF
    rpad_cols = jnp.minimum(iota + (W - 14), W - 1) + OFF
    third = jnp.float32(1.0 / 3.0)

    def img_body(t, carry):
        copies = []
        for r in range(BUFR):
            src_row = jnp.clip(s0 - 2 + r, 0, H - 1)
            cp = pltpu.make_async_copy(
                img_hbm.at[t, src_row], rows.at[r, pl.ds(OFF, W)], sem)
            cp.start()
            copies.append(cp)
        for cp in copies:
            cp.wait()

        def pad_row(r, c2):
            rvec = jnp.full((16,), r, jnp.int32)
            lv = plsc.load_gather(rows, [rvec, lpad_cols])
            rows[r, pl.ds(OFF - 2, 16)] = lv
            rv = plsc.load_gather(rows, [rvec, rpad_cols])
            rows[r, pl.ds(OFF + W - 14, 16)] = rv
            return c2
        lax.fori_loop(0, BUFR, pad_row, 0)

        def half_pass(half):
            def p_body(p, c2):
                pa = half * (RPW // 2) + p
                def j_body(j, c3):
                    cb = OFF + j * 16
                    a1156 = rows[pa + 2, pl.ds(cb, 16)] * jnp.float32(4 * L * L)
                    accs = [None, None, None, None]
                    for tid, (rb, cbb), (rc, ccc), rot in BRANCHES:
                        tab = (htab, dtab, btab)[tid]
                        bv = rows[pa + 2 + rb, pl.ds(cb + cbb, 16)]
                        cv = rows[pa + 2 + rc, pl.ds(cb + ccc, 16)]
                        idx4 = (a1156 + bv * jnp.float32(4 * L)
                                + cv * jnp.float32(4)).astype(jnp.int32)
                        perm = PERMS[rot]
                        for k in range(4):
                            c = perm[k]
                            g = plsc.load_gather(tab, [idx4 + c if c else idx4])
                            accs[k] = g if accs[k] is None else accs[k] + g
                    col0 = iota2 + j * 32
                    row0 = jnp.full((16,), 2 * p, jnp.int32)
                    row1 = row0 + 1
                    plsc.store_scatter(outb, [row0, col0], accs[0] * third)
                    plsc.store_scatter(outb, [row0, col0 + 1], accs[1] * third)
                    plsc.store_scatter(outb, [row1, col0], accs[2] * third)
                    plsc.store_scatter(outb, [row1, col0 + 1], accs[3] * third)
                    return c3
                lax.fori_loop(0, NG, j_body, 0)
                return c2
            lax.fori_loop(0, RPW // 2, p_body, 0)
            pltpu.sync_copy(
                outb, out_hbm.at[t, pl.ds(s0 * 2 + half * RPW, RPW)])
        half_pass(0)
        half_pass(1)
        return carry
    lax.fori_loop(0, NIMG, img_body, 0)


def kernel(img_lr, h_weight, d_weight, b_weight):
    img3 = img_lr.reshape(NIMG, H, W)
    out3 = _hdblut_sc(img3, h_weight.reshape(-1), d_weight.reshape(-1),
                      b_weight.reshape(-1))
    return out3.reshape(4, 3, 2 * H, 2 * W)


# speedup vs baseline: 583.9320x; 1.2403x over previous
import functools

import jax
import jax.numpy as jnp
from jax import lax
from jax.experimental import pallas as pl
from jax.experimental.pallas import tpu as pltpu
from jax.experimental.pallas import tpu_sc as plsc

L = 17
H = 512
W = 512
NIMG = 12
NWORK = 32
RPW = H // NWORK
BUFR = RPW + 4
OFF = 8
RS = 528
NG = W // 16
NPACK = L**3 * 2

BRANCHES = (
    (0, (0, 1), (0, 2), 0),
    (0, (1, 0), (2, 0), 1),
    (0, (0, -1), (0, -2), 2),
    (0, (-1, 0), (-2, 0), 3),
    (1, (1, 1), (2, 2), 0),
    (1, (1, -1), (2, -2), 1),
    (1, (-1, -1), (-2, -2), 2),
    (1, (-1, 1), (-2, 2), 3),
    (2, (1, 2), (2, 1), 0),
    (2, (2, -1), (1, -2), 1),
    (2, (-1, -2), (-2, -1), 2),
    (2, (-2, 1), (-1, 2), 3),
)

_mesh = plsc.VectorSubcoreMesh(core_axis_name="c", subcore_axis_name="s")

def _bf(x):
    return plsc.bitcast(x, jnp.bfloat16)


def _i32(x):
    return plsc.bitcast(x, jnp.int32)


@functools.partial(
    pl.kernel,
    out_type=jax.ShapeDtypeStruct((NIMG, 2 * H, 2 * W), jnp.float32),
    mesh=_mesh,
    scratch_types=[
        pltpu.VMEM((NPACK,), jnp.int32),
        pltpu.VMEM((NPACK,), jnp.int32),
        pltpu.VMEM((NPACK,), jnp.int32),
        pltpu.VMEM((NPACK,), jnp.int32),
        pltpu.VMEM((NPACK,), jnp.int32),
        pltpu.VMEM((NPACK,), jnp.int32),
        pltpu.VMEM((BUFR, RS), jnp.float32),
        pltpu.VMEM((BUFR, RS), jnp.float32),
        pltpu.VMEM((BUFR, RS), jnp.float32),
        pltpu.VMEM((RPW, 2 * W), jnp.float32),
        pltpu.SemaphoreType.DMA,
    ],
    compiler_params=pltpu.CompilerParams(
        use_tc_tiling_on_sc=False, needs_layout_passes=False),
)
def _hdblut_sc(img_hbm, h0_hbm, h1_hbm, d0_hbm, d1_hbm, b0_hbm, b1_hbm,
               out_hbm, h0, h1, d0, d1, b0, b1, raw, bbuf, cbuf, outb, sem):
    wid = lax.axis_index("c") * 16 + lax.axis_index("s")
    s0 = wid * RPW
    ptabs = (h0, h1, d0, d1, b0, b1)

    for src, dst in zip((h0_hbm, h1_hbm, d0_hbm, d1_hbm, b0_hbm, b1_hbm),
                        ptabs):
        pltpu.sync_copy(src, dst)

    iota = lax.iota(jnp.int32, 16)
    iota2 = iota * 2
    lpad_cols = jnp.maximum(iota - 2, 0) + OFF
    rpad_cols = jnp.minimum(iota + (W - 14), W - 1) + OFF
    third = jnp.float32(1.0 / 3.0)
    hi_mask = jnp.int32(-65536)

    def img_body(t, carry):
        copies = []
        for r in range(BUFR):
            src_row = jnp.clip(s0 - 2 + r, 0, H - 1)
            cp = pltpu.make_async_copy(
                img_hbm.at[t, src_row], raw.at[r, pl.ds(OFF, W)], sem)
            cp.start()
            copies.append(cp)
        for cp in copies:
            cp.wait()

        def conv_row(r, cc):
            def conv_grp(jj, c3):
                base = OFF + jj * 16
                vf = raw[r, pl.ds(base, 16)]
                bbuf[r, pl.ds(base, 16)] = vf * jnp.float32(2 * L)
                cbuf[r, pl.ds(base, 16)] = vf * jnp.float32(2)
                return c3
            lax.fori_loop(0, NG, conv_grp, 0)
            return cc
        lax.fori_loop(0, BUFR, conv_row, 0)

        def pad_row(r, cc):
            rvec = jnp.full((16,), r, jnp.int32)
            for buf in (bbuf, cbuf):
                lv = plsc.load_gather(buf, [rvec, lpad_cols])
                buf[r, pl.ds(OFF - 2, 16)] = lv
                rv = plsc.load_gather(buf, [rvec, rpad_cols])
                buf[r, pl.ds(OFF + W - 14, 16)] = rv
            return cc
        lax.fori_loop(0, BUFR, pad_row, 0)

        def half_pass(half):
            def p_body(p, cc):
                pa = half * (RPW // 2) + p
                def j_body(j, c3):
                    cb = OFF + j * 16
                    a578 = bbuf[pa + 2, pl.ds(cb, 16)] * jnp.float32(L)
                    accA = accB = accAr = accBr = None
                    for tid, (rb, cbb), (rc, ccc), rot in BRANCHES:
                        tab = ptabs[tid * 2 + (rot & 1)]
                        bv = bbuf[pa + 2 + rb, pl.ds(cb + cbb, 16)]
                        cv = cbuf[pa + 2 + rc, pl.ds(cb + ccc, 16)]
                        idx2 = (a578 + bv + cv).astype(jnp.int32)
                        gA = _bf(plsc.load_gather(tab, [idx2]))
                        gB = _bf(plsc.load_gather(tab, [idx2 + 1]))
                        if rot < 2:
                            accA = gA if accA is None else accA + gA
                            accB = gB if accB is None else accB + gB
                        else:
                            accAr = gB if accAr is None else accAr + gB
                            accBr = gA if accBr is None else accBr + gA
                    iAr, iBr = _i32(accAr), _i32(accBr)
                    sw0 = jnp.bitwise_or(lax.shift_left(iAr, 16),
                                         lax.shift_right_logical(iAr, 16))
                    sw1 = jnp.bitwise_or(lax.shift_left(iBr, 16),
                                         lax.shift_right_logical(iBr, 16))
                    acc0 = _i32(accA + _bf(sw0))
                    acc1 = _i32(accB + _bf(sw1))
                    o00 = plsc.bitcast(lax.shift_left(acc0, 16), jnp.float32)
                    o01 = plsc.bitcast(jnp.bitwise_and(acc0, hi_mask), jnp.float32)
                    o10 = plsc.bitcast(lax.shift_left(acc1, 16), jnp.float32)
                    o11 = plsc.bitcast(jnp.bitwise_and(acc1, hi_mask), jnp.float32)
                    col0 = iota2 + j * 32
                    row0 = jnp.full((16,), 2 * p, jnp.int32)
                    row1 = row0 + 1
                    plsc.store_scatter(outb, [row0, col0], o00 * third)
                    plsc.store_scatter(outb, [row0, col0 + 1], o01 * third)
                    plsc.store_scatter(outb, [row1, col0], o10 * third)
                    plsc.store_scatter(outb, [row1, col0 + 1], o11 * third)
                    return c3
                lax.fori_loop(0, NG, j_body, 0)
                return cc
            lax.fori_loop(0, RPW // 2, p_body, 0)
            pltpu.sync_copy(
                outb, out_hbm.at[t, pl.ds(s0 * 2 + half * RPW, RPW)])
        half_pass(0)
        half_pass(1)
        return carry
    lax.fori_loop(0, NIMG, img_body, 0)


def _pack_pairs(w, pairs):
    u = lax.bitcast_convert_type(
        w.astype(jnp.bfloat16), jnp.uint16).astype(jnp.uint32)
    cols = [u[:, lo] | (u[:, hi] << 16) for lo, hi in pairs]
    packed = jnp.stack(cols, axis=1).reshape(-1)
    return lax.bitcast_convert_type(packed, jnp.int32)


def kernel(img_lr, h_weight, d_weight, b_weight):
    img3 = img_lr.reshape(NIMG, H, W)
    p0 = [(0, 1), (2, 3)]
    p1 = [(2, 0), (3, 1)]
    tabs = []
    for w in (h_weight, d_weight, b_weight):
        tabs.append(_pack_pairs(w, p0))
        tabs.append(_pack_pairs(w, p1))
    out3 = _hdblut_sc(img3, *tabs)
    return out3.reshape(4, 3, 2 * H, 2 * W)


# speedup vs baseline: 1505.8931x; 2.5789x over previous
import functools

import jax
import jax.numpy as jnp
from jax import lax
from jax.experimental import pallas as pl
from jax.experimental.pallas import tpu as pltpu
from jax.experimental.pallas import tpu_sc as plsc

L = 17
H = 512
W = 512
NIMG = 12
NWORK = 32
RPW = H // NWORK
BUFR = RPW + 4
OFF = 8
RS = 528
NG = W // 16
NPACK = L**3 * 2

BRANCHES = (
    (0, (0, 1), (0, 2), 0),
    (0, (1, 0), (2, 0), 1),
    (0, (0, -1), (0, -2), 2),
    (0, (-1, 0), (-2, 0), 3),
    (1, (1, 1), (2, 2), 0),
    (1, (1, -1), (2, -2), 1),
    (1, (-1, -1), (-2, -2), 2),
    (1, (-1, 1), (-2, 2), 3),
    (2, (1, 2), (2, 1), 0),
    (2, (2, -1), (1, -2), 1),
    (2, (-1, -2), (-2, -1), 2),
    (2, (-2, 1), (-1, 2), 3),
)

_mesh = plsc.VectorSubcoreMesh(core_axis_name="c", subcore_axis_name="s")

def _bf(x):
    return plsc.bitcast(x, jnp.bfloat16)


def _i32(x):
    return plsc.bitcast(x, jnp.int32)


@functools.partial(
    pl.kernel,
    out_type=jax.ShapeDtypeStruct((NIMG, 2 * H, 2 * W), jnp.float32),
    mesh=_mesh,
    scratch_types=[
        pltpu.VMEM((NPACK,), jnp.int32),
        pltpu.VMEM((NPACK,), jnp.int32),
        pltpu.VMEM((NPACK,), jnp.int32),
        pltpu.VMEM((NPACK,), jnp.int32),
        pltpu.VMEM((NPACK,), jnp.int32),
        pltpu.VMEM((NPACK,), jnp.int32),
        pltpu.VMEM((BUFR, RS), jnp.float32),
        pltpu.VMEM((BUFR, RS), jnp.float32),
        pltpu.VMEM((BUFR, RS), jnp.float32),
        pltpu.VMEM((RPW, 2 * W), jnp.float32),
        pltpu.SemaphoreType.DMA,
    ],
    compiler_params=pltpu.CompilerParams(
        use_tc_tiling_on_sc=False, needs_layout_passes=False),
)
def _hdblut_sc(img_hbm, h0_hbm, h1_hbm, d0_hbm, d1_hbm, b0_hbm, b1_hbm,
               out_hbm, h0, h1, d0, d1, b0, b1, raw, bbuf, cbuf, outb, sem):
    wid = lax.axis_index("c") * 16 + lax.axis_index("s")
    s0 = wid * RPW
    ptabs = (h0, h1, d0, d1, b0, b1)

    for src, dst in zip((h0_hbm, h1_hbm, d0_hbm, d1_hbm, b0_hbm, b1_hbm),
                        ptabs):
        pltpu.sync_copy(src, dst)

    iota = lax.iota(jnp.int32, 16)
    iota2 = iota * 2
    lpad_cols = jnp.maximum(iota - 2, 0) + OFF
    rpad_cols = jnp.minimum(iota + (W - 14), W - 1) + OFF
    third = jnp.float32(1.0 / 3.0)
    hi_mask = jnp.int32(-65536)

    def img_body(t, carry):
        copies = []
        for r in range(BUFR):
            src_row = jnp.clip(s0 - 2 + r, 0, H - 1)
            cp = pltpu.make_async_copy(
                img_hbm.at[t, src_row], raw.at[r, pl.ds(OFF, W)], sem)
            cp.start()
            copies.append(cp)
        for cp in copies:
            cp.wait()

        def conv_row(r, cc):
            def conv_grp(jj, c3):
                base = OFF + jj * 16
                vf = raw[r, pl.ds(base, 16)]
                bbuf[r, pl.ds(base, 16)] = vf * jnp.float32(2 * L)
                cbuf[r, pl.ds(base, 16)] = vf * jnp.float32(2)
                return c3
            lax.fori_loop(0, NG, conv_grp, 0)
            return cc
        lax.fori_loop(0, BUFR, conv_row, 0)

        def pad_row(r, cc):
            rvec = jnp.full((16,), r, jnp.int32)
            for buf in (bbuf, cbuf):
                lv = plsc.load_gather(buf, [rvec, lpad_cols])
                buf[r, pl.ds(OFF - 2, 16)] = lv
                rv = plsc.load_gather(buf, [rvec, rpad_cols])
                buf[r, pl.ds(OFF + W - 14, 16)] = rv
            return cc
        lax.fori_loop(0, BUFR, pad_row, 0)

        def half_pass(half):
            def p_body(p, cc):
                pa = half * (RPW // 2) + p
                @functools.partial(plsc.parallel_loop, 0, NG, unroll=2)
                def j_body(j):
                    cb = OFF + j * 16
                    a578 = bbuf[pa + 2, pl.ds(cb, 16)] * jnp.float32(L)
                    accA = accB = accAr = accBr = None
                    for tid, (rb, cbb), (rc, ccc), rot in BRANCHES:
                        tab = ptabs[tid * 2 + (rot & 1)]
                        bv = bbuf[pa + 2 + rb, pl.ds(cb + cbb, 16)]
                        cv = cbuf[pa + 2 + rc, pl.ds(cb + ccc, 16)]
                        idx2 = (a578 + bv + cv).astype(jnp.int32)
                        gA = _bf(plsc.load_gather(tab, [idx2]))
                        gB = _bf(plsc.load_gather(tab, [idx2 + 1]))
                        if rot < 2:
                            accA = gA if accA is None else accA + gA
                            accB = gB if accB is None else accB + gB
                        else:
                            accAr = gB if accAr is None else accAr + gB
                            accBr = gA if accBr is None else accBr + gA
                    iAr, iBr = _i32(accAr), _i32(accBr)
                    sw0 = jnp.bitwise_or(lax.shift_left(iAr, 16),
                                         lax.shift_right_logical(iAr, 16))
                    sw1 = jnp.bitwise_or(lax.shift_left(iBr, 16),
                                         lax.shift_right_logical(iBr, 16))
                    acc0 = _i32(accA + _bf(sw0))
                    acc1 = _i32(accB + _bf(sw1))
                    o00 = plsc.bitcast(lax.shift_left(acc0, 16), jnp.float32)
                    o01 = plsc.bitcast(jnp.bitwise_and(acc0, hi_mask), jnp.float32)
                    o10 = plsc.bitcast(lax.shift_left(acc1, 16), jnp.float32)
                    o11 = plsc.bitcast(jnp.bitwise_and(acc1, hi_mask), jnp.float32)
                    col0 = iota2 + j * 32
                    row0 = jnp.full((16,), 2 * p, jnp.int32)
                    row1 = row0 + 1
                    plsc.store_scatter(outb, [row0, col0], o00 * third)
                    plsc.store_scatter(outb, [row0, col0 + 1], o01 * third)
                    plsc.store_scatter(outb, [row1, col0], o10 * third)
                    plsc.store_scatter(outb, [row1, col0 + 1], o11 * third)
                return cc
            lax.fori_loop(0, RPW // 2, p_body, 0)
            pltpu.sync_copy(
                outb, out_hbm.at[t, pl.ds(s0 * 2 + half * RPW, RPW)])
        half_pass(0)
        half_pass(1)
        return carry
    lax.fori_loop(0, NIMG, img_body, 0)


def _pack_pairs(w, pairs):
    u = lax.bitcast_convert_type(
        w.astype(jnp.bfloat16), jnp.uint16).astype(jnp.uint32)
    cols = [u[:, lo] | (u[:, hi] << 16) for lo, hi in pairs]
    packed = jnp.stack(cols, axis=1).reshape(-1)
    return lax.bitcast_convert_type(packed, jnp.int32)


def kernel(img_lr, h_weight, d_weight, b_weight):
    img3 = img_lr.reshape(NIMG, H, W)
    p0 = [(0, 1), (2, 3)]
    p1 = [(2, 0), (3, 1)]
    tabs = []
    for w in (h_weight, d_weight, b_weight):
        tabs.append(_pack_pairs(w, p0))
        tabs.append(_pack_pairs(w, p1))
    out3 = _hdblut_sc(img3, *tabs)
    return out3.reshape(4, 3, 2 * H, 2 * W)
